# Initial kernel scaffold; baseline (speedup 1.0000x reference)
import jax
import jax.numpy as jnp
from jax.experimental import pallas as pl


def kernel(x, w_kn, b):
    raise NotImplementedError("write your pallas kernel here")


# speedup vs baseline: 3.0330x; 3.0330x over previous
import jax
import jax.numpy as jnp
from jax.experimental import pallas as pl
from jax.experimental.pallas import tpu as pltpu


def _round_up(v, m):
    return ((v + m - 1) // m) * m


def _dense_kernel(x_ref, w_ref, b_ref, o_ref):
    acc = jnp.dot(x_ref[...], w_ref[...], preferred_element_type=jnp.float32)
    o_ref[...] = (acc + b_ref[...].astype(jnp.float32)).astype(o_ref.dtype)


def kernel(x, w_kn, b):
    in_dim, out_dim = w_kn.shape
    orig_shape = x.shape
    out_dtype = x.dtype

    x2 = x.reshape(-1, in_dim)
    m = x2.shape[0]

    k_p = _round_up(in_dim, 128)
    n_p = _round_up(out_dim, 128)
    w_p = w_kn
    if (k_p, n_p) != (in_dim, out_dim):
        w_p = jnp.pad(w_kn, ((0, k_p - in_dim), (0, n_p - out_dim)))
    b_p = b
    if b.shape != (1, n_p):
        b_p = jnp.pad(b.reshape(1, -1), ((0, 0), (0, n_p - b.size)))

    tm = min(512, _round_up(m, 8))
    m_p = _round_up(m, tm)
    x_p = x2
    if (m_p, k_p) != (m, in_dim):
        x_p = jnp.pad(x2, ((0, m_p - m), (0, k_p - in_dim)))

    grid = (m_p // tm,)
    x_item = jnp.dtype(x_p.dtype).itemsize
    o_item = jnp.dtype(out_dtype).itemsize
    cost = pl.CostEstimate(
        flops=2 * m_p * k_p * n_p,
        transcendentals=0,
        bytes_accessed=(m_p * k_p * x_item + k_p * n_p * 4
                        + n_p * 4 + m_p * n_p * o_item),
    )

    out_p = pl.pallas_call(
        _dense_kernel,
        out_shape=jax.ShapeDtypeStruct((m_p, n_p), out_dtype),
        grid=grid,
        in_specs=[
            pl.BlockSpec((tm, k_p), lambda i: (i, 0)),
            pl.BlockSpec((k_p, n_p), lambda i: (0, 0)),
            pl.BlockSpec((1, n_p), lambda i: (0, 0)),
        ],
        out_specs=pl.BlockSpec((tm, n_p), lambda i: (i, 0)),
        compiler_params=pltpu.CompilerParams(
            dimension_semantics=("parallel",),
            vmem_limit_bytes=60 * 1024 * 1024,
        ),
        cost_estimate=cost,
    )(x_p, w_p, b_p)

    out = out_p[:m, :out_dim]
    return out.reshape(orig_shape[:-1] + (out_dim,))
